# Initial kernel scaffold; baseline (speedup 1.0000x reference)
import jax
import jax.numpy as jnp
from jax.experimental import pallas as pl


def kernel(p_ext, R, r, j_idx, lambda_raw):
    raise NotImplementedError("write your pallas kernel here")


# speedup vs baseline: 264.2678x; 264.2678x over previous
import functools

import jax
import jax.numpy as jnp
from jax import lax
from jax.experimental import pallas as pl
from jax.experimental.pallas import tpu as pltpu
from jax.experimental.pallas import tpu_sc as plsc

MU1, SIGMA1, MU2, SIGMA2 = 5.79, 0.87, 10.68, 1.78
MIN_SEQ_SEP = 5
MAX_DIST = 12.0
TAU_SQ = 0.02 ** 2

NC, NS, LANES = 2, 16, 16
NW = NC * NS


def _sc_partials(p_full, r2, j2, B, L, K):
    rows = L // NW
    ch = rows * K
    vecs_per_row = K // LANES

    mesh = plsc.VectorSubcoreMesh(
        core_axis_name="c", subcore_axis_name="s",
        num_cores=NC, num_subcores=NS)

    @functools.partial(
        pl.kernel,
        out_type=jax.ShapeDtypeStruct((NW, B, LANES), jnp.float32),
        mesh=mesh,
        compiler_params=pltpu.CompilerParams(needs_layout_passes=False),
        scratch_types=[
            pltpu.VMEM((L,), jnp.float32),
            pltpu.VMEM((ch,), jnp.float32),
            pltpu.VMEM((ch,), jnp.int32),
            pltpu.VMEM((LANES,), jnp.float32),
        ],
    )
    def k(pf_hbm, r_hbm, j_hbm, out_hbm, table, rv, jv, accv):
        cid = lax.axis_index("c")
        sid = lax.axis_index("s")
        wid = sid * NC + cid
        row0 = wid * rows
        base = pl.multiple_of(wid * ch, ch)

        def batch_body(b, _):
            pltpu.sync_copy(pf_hbm.at[b], table)
            pltpu.sync_copy(r_hbm.at[b, pl.ds(base, ch)], rv)
            pltpu.sync_copy(j_hbm.at[b, pl.ds(base, ch)], jv)

            def row_body(rr, acc):
                l = row0 + rr
                l_vec = jnp.full((LANES,), l, jnp.int32)
                p_i = plsc.load_gather(table, [l_vec])
                for c in range(vecs_per_row):
                    off = pl.multiple_of(rr * K + c * LANES, LANES)
                    jvec = jv[pl.ds(off, LANES)]
                    rvec = rv[pl.ds(off, LANES)]
                    valid = rvec < (MAX_DIST - 0.0001)
                    sep_ok = jnp.abs(jvec - l_vec) > MIN_SEQ_SEP
                    mask = jnp.logical_and(valid, sep_ok)
                    rc = jnp.minimum(rvec, MAX_DIST)
                    z1 = (rc - MU1) * (1.0 / SIGMA1)
                    z2 = (rc - MU2) * (1.0 / SIGMA2)
                    g = jnp.exp(-0.5 * z1 * z1) + jnp.exp(-0.5 * z2 * z2)
                    p_j = plsc.load_gather(table, [jvec])
                    s = (p_i * p_j) * g
                    s = jnp.where(mask, s, 0.0)
                    s2 = s * s
                    acc = acc + s * s2 / (s2 + TAU_SQ)
                return acc

            acc = lax.fori_loop(0, rows, row_body,
                                jnp.zeros((LANES,), jnp.float32))
            accv[...] = acc
            pltpu.sync_copy(accv, out_hbm.at[wid, b])
            return 0

        lax.fori_loop(0, B, batch_body, 0)

    return k(p_full, r2, j2)


def kernel(p_ext, R, r, j_idx, lambda_raw):
    del R
    B, L, K = r.shape
    p_full = jnp.pad(p_ext, ((0, 0), (1, 0)))
    r2 = r.reshape(B, L * K)
    j2 = j_idx.reshape(B, L * K)
    partials = _sc_partials(p_full, r2, j2, B, L, K)
    e_sum = partials.sum(axis=(0, 2))
    lambda_hb = jax.nn.softplus(lambda_raw) + 1e-06
    return -lambda_hb * e_sum / float(max(L, 1))


# speedup vs baseline: 413.9340x; 1.5663x over previous
import functools

import jax
import jax.numpy as jnp
from jax import lax
from jax.experimental import pallas as pl
from jax.experimental.pallas import tpu as pltpu
from jax.experimental.pallas import tpu_sc as plsc

MU1, SIGMA1, MU2, SIGMA2 = 5.79, 0.87, 10.68, 1.78
MIN_SEQ_SEP = 5
MAX_DIST = 12.0
TAU_SQ = 0.02 ** 2

NC, NS, LANES = 2, 16, 16
NW = NC * NS


def _sc_partials(p_full, r, j_idx, B, L, K):
    rows = L // NW
    vecs_per_row = K // LANES

    mesh = plsc.VectorSubcoreMesh(
        core_axis_name="c", subcore_axis_name="s",
        num_cores=NC, num_subcores=NS)

    @functools.partial(
        pl.kernel,
        out_type=jax.ShapeDtypeStruct((NW, B, LANES), jnp.float32),
        mesh=mesh,
        compiler_params=pltpu.CompilerParams(needs_layout_passes=False),
        scratch_types=[
            pltpu.VMEM((L,), jnp.float32),
            pltpu.VMEM((L,), jnp.float32),
            pltpu.VMEM((2, rows, K), jnp.float32),
            pltpu.VMEM((2, rows, K), jnp.int32),
            pltpu.VMEM((B, LANES), jnp.float32),
            pltpu.SemaphoreType.DMA,
            pltpu.SemaphoreType.DMA,
        ],
    )
    def k(pf_hbm, r_hbm, j_hbm, out_hbm, table0, table1, rv, jv, accv,
          sem0, sem1):
        tables = (table0, table1)
        cid = lax.axis_index("c")
        sid = lax.axis_index("s")
        wid = sid * NC + cid
        row0 = wid * rows
        sems = (sem0, sem1)

        def start_batch(b, slot):
            pltpu.async_copy(pf_hbm.at[b], tables[slot], sems[slot])
            pltpu.async_copy(r_hbm.at[b, pl.ds(row0, rows)],
                             rv.at[slot], sems[slot])
            pltpu.async_copy(j_hbm.at[b, pl.ds(row0, rows)],
                             jv.at[slot], sems[slot])

        def wait_slot(slot):
            pltpu.make_async_copy(pf_hbm.at[0], tables[slot],
                                  sems[slot]).wait()
            pltpu.make_async_copy(r_hbm.at[0, pl.ds(0, rows)],
                                  rv.at[slot], sems[slot]).wait()
            pltpu.make_async_copy(j_hbm.at[0, pl.ds(0, rows)],
                                  jv.at[slot], sems[slot]).wait()

        def compute_batch(b, slot):
            table = tables[slot]

            def row_body(rr, acc):
                l = row0 + rr
                l_vec = jnp.full((LANES,), l, jnp.int32)
                p_i = plsc.load_gather(table, [l_vec])
                for c in range(vecs_per_row):
                    off = pl.multiple_of(c * LANES, LANES)
                    jvec = jv[slot, rr, pl.ds(off, LANES)]
                    rvec = rv[slot, rr, pl.ds(off, LANES)]
                    valid = rvec < (MAX_DIST - 0.0001)
                    sep_ok = (jvec - l_vec + MIN_SEQ_SEP).astype(jnp.uint32) \
                        > (2 * MIN_SEQ_SEP)
                    mask = jnp.logical_and(valid, sep_ok)
                    z1 = (rvec - MU1) * (1.0 / SIGMA1)
                    z2 = (rvec - MU2) * (1.0 / SIGMA2)
                    g = jnp.exp(-0.5 * z1 * z1) + jnp.exp(-0.5 * z2 * z2)
                    p_j = plsc.load_gather(table, [jvec])
                    s = (p_i * p_j) * g
                    s = jnp.where(mask, s, 0.0)
                    s2 = s * s
                    acc = acc + s * s2 / (s2 + TAU_SQ)
                return acc

            acc = lax.fori_loop(0, rows, row_body,
                                jnp.zeros((LANES,), jnp.float32))
            accv[b] = acc

        start_batch(0, 0)

        def pair_body(t, _):
            b = 2 * t
            start_batch(b + 1, 1)
            wait_slot(0)
            compute_batch(b, 0)

            @pl.when(b + 2 < B)
            def _():
                start_batch(b + 2, 0)

            wait_slot(1)
            compute_batch(b + 1, 1)
            return 0

        lax.fori_loop(0, B // 2, pair_body, 0)
        pltpu.sync_copy(accv, out_hbm.at[wid])

    return k(p_full, r, j_idx)


def kernel(p_ext, R, r, j_idx, lambda_raw):
    del R
    B, L, K = r.shape
    p_full = jnp.pad(p_ext, ((0, 0), (1, 0)))
    partials = _sc_partials(p_full, r, j_idx, B, L, K)
    e_sum = partials.sum(axis=(0, 2))
    lambda_hb = jax.nn.softplus(lambda_raw) + 1e-06
    return -lambda_hb * e_sum / float(max(L, 1))
